# baseline (speedup 1.0000x reference)
import jax
import jax.numpy as jnp
from jax import lax
from jax.experimental import pallas as pl
from jax.experimental.pallas import tpu as pltpu
from jax.experimental.pallas import tpu_sc as plsc

_VOCAB = 100000
_VPAD = 102400
_EMB = 32
_B = 4096
_NT = 25
_NTA = 13
_NTB = _NT - _NTA
_NFA = _NTA + 8
_NFB = _NTB
_NC = 2
_NS = 16
_NW = _NC * _NS
_RPW = _B // _NW
_HALF = 16


def _score2_body(w_ref, t_ref, of_ref, oa_ref):
    s = lax.dot_general(w_ref[0], t_ref[0], (((1,), (0,)), ((), ())),
                        preferred_element_type=jnp.float32)
    of_ref[...] = s[0]
    oa_ref[...] = s[1]


def _score1_body(w_ref, t_ref, of_ref):
    s = lax.dot_general(w_ref[0], t_ref[0], (((1,), (0,)), ((), ())),
                        preferred_element_type=jnp.float32)
    of_ref[...] = s[0]


def _sc_partial_body(g_hbm, of_hbm, oa_hbm, out_hbm,
                     gidx_v, sbuf_v, outv_v, sem):
    wid = lax.axis_index("s") * _NC + lax.axis_index("c")
    pltpu.sync_copy(g_hbm.at[wid], gidx_v)
    for j in range(_NFA):
        src = of_hbm if j < _NTA else oa_hbm
        pltpu.async_copy(src.at[gidx_v.at[j]],
                         sbuf_v.at[pl.ds(j * _RPW, _RPW)], sem)
    pltpu.make_async_copy(of_hbm.at[pl.ds(0, _NFA * _RPW)], sbuf_v, sem).wait()
    for g in range(_RPW // _HALF):
        tot = sbuf_v[pl.ds(g * _HALF, _HALF)]
        for j in range(1, _NFA):
            tot = tot + sbuf_v[pl.ds(j * _RPW + g * _HALF, _HALF)]
        outv_v[pl.ds(g * _HALF, _HALF)] = tot
    pltpu.sync_copy(outv_v, out_hbm.at[pl.ds(wid * _RPW, _RPW)])


def _sc_final_body(g_hbm, of_hbm, p1_hbm, bias_hbm, out_hbm,
                   gidx_v, sbuf_v, p1_v, outv_v, bias_v, sem):
    wid = lax.axis_index("s") * _NC + lax.axis_index("c")
    pltpu.sync_copy(bias_hbm, bias_v)
    pltpu.sync_copy(g_hbm.at[wid], gidx_v)
    for j in range(_NFB):
        pltpu.async_copy(of_hbm.at[gidx_v.at[j]],
                         sbuf_v.at[pl.ds(j * _RPW, _RPW)], sem)
    pltpu.sync_copy(p1_hbm.at[pl.ds(wid * _RPW, _RPW)], p1_v)
    pltpu.make_async_copy(of_hbm.at[pl.ds(0, _NFB * _RPW)], sbuf_v, sem).wait()
    bias = bias_v[:]
    for g in range(_RPW // _HALF):
        tot = p1_v[pl.ds(g * _HALF, _HALF)]
        for j in range(_NFB):
            tot = tot + sbuf_v[pl.ds(j * _RPW + g * _HALF, _HALF)]
        tot = tot + bias
        p = 1.0 / (1.0 + jnp.exp(-tot))
        outv_v[pl.ds(g * _HALF, _HALF)] = p
    pltpu.sync_copy(outv_v, out_hbm.at[pl.ds(wid * _RPW, _RPW)])


def _sc_gather_kernel(body, out_shape, scratch, args):
    mesh = plsc.VectorSubcoreMesh(core_axis_name="c", subcore_axis_name="s")
    return pl.kernel(
        body,
        out_type=jax.ShapeDtypeStruct(out_shape, jnp.float32),
        mesh=mesh,
        compiler_params=pltpu.CompilerParams(needs_layout_passes=False,
                                             use_tc_tiling_on_sc=False),
        scratch_types=scratch,
    )(*args)


def kernel(ad_feature_inputs, feature_inputs, tables, W_meta, b_meta,
           W_pred, b_pred):
    t_t = tables.transpose(0, 2, 1)

    w0 = W_pred[0, :_EMB]
    v = W_meta.T @ w0
    w_feat = W_pred[0, _EMB:].reshape(_NT, _EMB)
    w_ad = jnp.zeros((_NTA, _EMB), jnp.float32).at[:8].set(
        jnp.broadcast_to((v / _EMB)[:, None], (8, _EMB)))
    w_a = jnp.stack([w_feat[:_NTA], w_ad], axis=1)
    w_b = w_feat[_NTA:].reshape(_NTB, 1, _EMB)
    bias_c = b_pred[0] + jnp.dot(b_meta, w0)
    bias_vec = jnp.full((_HALF,), bias_c, jnp.float32)

    of_a, oa_a = pl.pallas_call(
        _score2_body,
        grid=(_NTA,),
        in_specs=[
            pl.BlockSpec((1, 2, _EMB), lambda t: (t, 0, 0)),
            pl.BlockSpec((1, _EMB, _VPAD), lambda t: (t + 1, 0, 0)),
        ],
        out_specs=[
            pl.BlockSpec((_VPAD,), lambda t: (t,)),
            pl.BlockSpec((_VPAD,), lambda t: (t,)),
        ],
        out_shape=[
            jax.ShapeDtypeStruct((_NTA * _VPAD,), jnp.float32),
            jax.ShapeDtypeStruct((_NTA * _VPAD,), jnp.float32),
        ],
    )(w_a, t_t)

    of_b = pl.pallas_call(
        _score1_body,
        grid=(_NTB,),
        in_specs=[
            pl.BlockSpec((1, 1, _EMB), lambda t: (t, 0, 0)),
            pl.BlockSpec((1, _EMB, _VPAD), lambda t: (t + 1 + _NTA, 0, 0)),
        ],
        out_specs=pl.BlockSpec((_VPAD,), lambda t: (t,)),
        out_shape=jax.ShapeDtypeStruct((_NTB * _VPAD,), jnp.float32),
    )(w_b, t_t)

    offs = jnp.arange(_NT, dtype=jnp.int32) * _VPAD
    g_a = jnp.concatenate(
        [feature_inputs[:, :_NTA] + offs[None, :_NTA],
         ad_feature_inputs + offs[None, :8]], axis=1)
    g_a = g_a.reshape(_NW, _RPW, _NFA).transpose(0, 2, 1)
    g_b = feature_inputs[:, _NTA:] + offs[None, :_NTB]
    g_b = g_b.reshape(_NW, _RPW, _NFB).transpose(0, 2, 1)

    p1 = _sc_gather_kernel(
        _sc_partial_body, (_B,),
        [
            pltpu.VMEM((_NFA, _RPW), jnp.int32),
            pltpu.VMEM((_NFA * _RPW,), jnp.float32),
            pltpu.VMEM((_RPW,), jnp.float32),
            pltpu.SemaphoreType.DMA,
        ],
        (g_a, of_a, oa_a))
    out = _sc_gather_kernel(
        _sc_final_body, (_B,),
        [
            pltpu.VMEM((_NFB, _RPW), jnp.int32),
            pltpu.VMEM((_NFB * _RPW,), jnp.float32),
            pltpu.VMEM((_RPW,), jnp.float32),
            pltpu.VMEM((_RPW,), jnp.float32),
            pltpu.VMEM((_HALF,), jnp.float32),
            pltpu.SemaphoreType.DMA,
        ],
        (g_b, of_b, p1, bias_vec))
    return out[:, None]


# speedup vs baseline: 1.0051x; 1.0051x over previous
import jax
import jax.numpy as jnp
from jax import lax
from jax.experimental import pallas as pl
from jax.experimental.pallas import tpu as pltpu
from jax.experimental.pallas import tpu_sc as plsc

_VOCAB = 100000
_VPAD = 102400
_EMB = 32
_B = 4096
_NT = 25
_NF = 33
_NC = 2
_NS = 16
_NW = _NC * _NS
_RPW = _B // _NW
_HALF = 16


def _score_body(w_ref, t_ref, of_ref, oa_ref):
    s = lax.dot_general(w_ref[0], t_ref[0], (((1,), (0,)), ((), ())),
                        preferred_element_type=jnp.float32)
    of_ref[...] = s[0]
    oa_ref[...] = s[1]


def _sc_body(ft_hbm, ad_hbm, of_hbm, oa_hbm, bias_hbm, out_hbm,
             gidx_v, sbuf_v, outv_v, bias_v, isem, sem):
    wid = lax.axis_index("s") * _NC + lax.axis_index("c")
    base = wid * _RPW
    pltpu.sync_copy(bias_hbm, bias_v)
    for j in range(_NT):
        pltpu.async_copy(ft_hbm.at[pl.ds(j * _B + base, _RPW)],
                         gidx_v.at[j], isem)
    for i in range(8):
        pltpu.async_copy(ad_hbm.at[pl.ds(i * _B + base, _RPW)],
                         gidx_v.at[_NT + i], isem)
    pltpu.make_async_copy(ft_hbm.at[pl.ds(0, _NF * _RPW)], gidx_v, isem).wait()
    for j in range(_NT):
        pltpu.async_copy(of_hbm.at[pl.ds(j * _VPAD, _VPAD)].at[gidx_v.at[j]],
                         sbuf_v.at[pl.ds(j * _RPW, _RPW)], sem)
    for i in range(8):
        pltpu.async_copy(
            oa_hbm.at[pl.ds(i * _VPAD, _VPAD)].at[gidx_v.at[_NT + i]],
            sbuf_v.at[pl.ds((_NT + i) * _RPW, _RPW)], sem)
    pltpu.make_async_copy(of_hbm.at[pl.ds(0, _NF * _RPW)], sbuf_v, sem).wait()

    bias = bias_v[:]
    for g in range(_RPW // _HALF):
        tot = sbuf_v[pl.ds(g * _HALF, _HALF)]
        for j in range(1, _NF):
            tot = tot + sbuf_v[pl.ds(j * _RPW + g * _HALF, _HALF)]
        tot = tot + bias
        p = 1.0 / (1.0 + jnp.exp(-tot))
        outv_v[pl.ds(g * _HALF, _HALF)] = p
    pltpu.sync_copy(outv_v, out_hbm.at[pl.ds(base, _RPW)])


def kernel(ad_feature_inputs, feature_inputs, tables, W_meta, b_meta,
           W_pred, b_pred):
    t_t = tables.transpose(0, 2, 1)

    w0 = W_pred[0, :_EMB]
    v = W_meta.T @ w0
    w_feat = W_pred[0, _EMB:].reshape(_NT, _EMB)
    w_ad = jnp.zeros((_NT, _EMB), jnp.float32).at[:8].set(
        jnp.broadcast_to((v / _EMB)[:, None], (8, _EMB)))
    w_all = jnp.stack([w_feat, w_ad], axis=1)
    bias_c = b_pred[0] + jnp.dot(b_meta, w0)
    bias_vec = jnp.full((_HALF,), bias_c, jnp.float32)

    o_feat, o_ad = pl.pallas_call(
        _score_body,
        grid=(_NT,),
        in_specs=[
            pl.BlockSpec((1, 2, _EMB), lambda t: (t, 0, 0)),
            pl.BlockSpec((1, _EMB, _VPAD), lambda t: (t + 1, 0, 0)),
        ],
        out_specs=[
            pl.BlockSpec((_VPAD,), lambda t: (t,)),
            pl.BlockSpec((_VPAD,), lambda t: (t,)),
        ],
        out_shape=[
            jax.ShapeDtypeStruct((_NT * _VPAD,), jnp.float32),
            jax.ShapeDtypeStruct((_NT * _VPAD,), jnp.float32),
        ],
    )(w_all, t_t)

    ft1 = feature_inputs.T.reshape(_NT * _B)
    ad1 = ad_feature_inputs.T.reshape(8 * _B)

    mesh = plsc.VectorSubcoreMesh(core_axis_name="c", subcore_axis_name="s")
    out = pl.kernel(
        _sc_body,
        out_type=jax.ShapeDtypeStruct((_B,), jnp.float32),
        mesh=mesh,
        compiler_params=pltpu.CompilerParams(needs_layout_passes=False,
                                             use_tc_tiling_on_sc=False),
        scratch_types=[
            pltpu.VMEM((_NF, _RPW), jnp.int32),
            pltpu.VMEM((_NF * _RPW,), jnp.float32),
            pltpu.VMEM((_RPW,), jnp.float32),
            pltpu.VMEM((_HALF,), jnp.float32),
            pltpu.SemaphoreType.DMA,
            pltpu.SemaphoreType.DMA,
        ],
    )(ft1, ad1, o_feat, o_ad, bias_vec)
    return out[:, None]
